# baseline (speedup 1.0000x reference)
import functools

import jax
import jax.numpy as jnp
from jax import lax
from jax.experimental import pallas as pl
from jax.experimental.pallas import tpu as pltpu
from jax.experimental.pallas import tpu_sc as plsc

FIELD = 26
VOCAB = 100000
EMB = 32
BATCH = 4096
D1 = 32
D2 = 32

NW = 32
RPW = BATCH // NW
FPW = RPW * FIELD
CROWS = 8
GCH = CROWS * FIELD
NCHUNK = RPW // CROWS
NBUF = 3


def _sc_body(gidx_hbm, fidx_hbm, off_hbm, xv_hbm, so_hbm, fo_hbm,
             s_hbm, q_hbm, fop_hbm,
             gidx_v, fidx_v, off_v, xv_v, fo_v, s_v, q_v,
             emb0, emb1, emb2, sem0, sem1, sem2, sem_f):
    wid = lax.axis_index("s") * 2 + lax.axis_index("c")
    bufs = (emb0, emb1, emb2)
    sems = (sem0, sem1, sem2)

    pltpu.sync_copy(gidx_hbm.at[wid], gidx_v)
    pltpu.sync_copy(fidx_hbm.at[wid], fidx_v)
    pltpu.sync_copy(off_hbm.at[wid], off_v)
    pltpu.sync_copy(xv_hbm.at[wid], xv_v)

    fo_cps = []
    for c in range(NCHUNK):
        for sub in range(2):
            fo_cps.append(pltpu.async_copy(
                fo_hbm.at[fidx_v.at[c, sub]],
                fo_v.at[pl.ds(c * GCH + sub * 104, 104)], sem_f))

    def fire(c):
        b = bufs[c % NBUF]
        return [
            pltpu.async_copy(so_hbm.at[gidx_v.at[c, 0]],
                             b.at[pl.ds(0, 104)], sems[c % NBUF]),
            pltpu.async_copy(so_hbm.at[gidx_v.at[c, 1]],
                             b.at[pl.ds(104, 104)], sems[c % NBUF]),
        ]

    descs = {}
    for c in range(NBUF):
        descs[c] = fire(c)

    def make_row_body(c):
        b = bufs[c % NBUF]

        def row_body(r, carry):
            base = r * FIELD
            gbase = c * GCH + base
            xa = xv_v[pl.ds(gbase, 16)]
            xb = xv_v[pl.ds(gbase + FIELD - 16, 16)]
            oa = off_v[pl.ds(gbase, 16)]
            ob = off_v[pl.ds(gbase + FIELD - 16, 16)]
            s0 = jnp.zeros((16,), jnp.float32)
            s1 = jnp.zeros((16,), jnp.float32)
            q0 = jnp.zeros((16,), jnp.float32)
            q1 = jnp.zeros((16,), jnp.float32)
            for f in range(FIELD):
                if f < 16:
                    xv, off = xa[f], oa[f]
                else:
                    xv, off = xb[f - (FIELD - 16)], ob[f - (FIELD - 16)]
                t0 = b[base + f, pl.ds(off, 16)] * xv
                t1 = b[base + f, pl.ds(off + 16, 16)] * xv
                s0 += t0
                s1 += t1
                q0 += t0 * t0
                q1 += t1 * t1
            row = c * CROWS + r
            s_v[row, pl.ds(0, 16)] = s0
            s_v[row, pl.ds(16, 16)] = s1
            q_v[row, pl.ds(0, 16)] = q0
            q_v[row, pl.ds(16, 16)] = q1
            return carry

        return row_body

    for c in range(NCHUNK):
        for d in descs.pop(c):
            d.wait()
        lax.fori_loop(0, CROWS, make_row_body(c), 0)
        if c + NBUF < NCHUNK:
            descs[c + NBUF] = fire(c + NBUF)

    for d in fo_cps:
        d.wait()

    def fo_body(k, carry):
        off = k * 16
        fo_v[pl.ds(off, 16)] = fo_v[pl.ds(off, 16)] * xv_v[pl.ds(off, 16)]
        return carry

    lax.fori_loop(0, FPW // 16, fo_body, 0)

    pltpu.sync_copy(s_v, s_hbm.at[pl.ds(wid * RPW, RPW)])
    pltpu.sync_copy(q_v, q_hbm.at[pl.ds(wid * RPW, RPW)])
    pltpu.sync_copy(fo_v, fop_hbm.at[wid])


_sc_kernel = functools.partial(
    pl.kernel,
    out_type=[
        jax.ShapeDtypeStruct((BATCH, EMB), jnp.float32),
        jax.ShapeDtypeStruct((BATCH, EMB), jnp.float32),
        jax.ShapeDtypeStruct((NW, FPW), jnp.float32),
    ],
    mesh=plsc.VectorSubcoreMesh(core_axis_name="c", subcore_axis_name="s"),
    compiler_params=pltpu.CompilerParams(use_tc_tiling_on_sc=False),
    scratch_types=[
        pltpu.VMEM((NCHUNK, 2, 104), jnp.int32),
        pltpu.VMEM((NCHUNK, 2, 104), jnp.int32),
        pltpu.VMEM((FPW,), jnp.int32),
        pltpu.VMEM((FPW,), jnp.float32),
        pltpu.VMEM((FPW,), jnp.float32),
        pltpu.VMEM((RPW, EMB), jnp.float32),
        pltpu.VMEM((RPW, EMB), jnp.float32),
        pltpu.VMEM((GCH, 128), jnp.float32),
        pltpu.VMEM((GCH, 128), jnp.float32),
        pltpu.VMEM((GCH, 128), jnp.float32),
        pltpu.SemaphoreType.DMA,
        pltpu.SemaphoreType.DMA,
        pltpu.SemaphoreType.DMA,
        pltpu.SemaphoreType.DMA,
    ],
)(_sc_body)


def _tc_body(s_ref, q_ref, fop_ref, w1_ref, b1_ref, w2_ref, b2_ref,
             bias_ref, out_ref):
    s = s_ref[...]
    x = 0.5 * (s * s - q_ref[...])
    h = jnp.maximum(
        jnp.dot(x, w1_ref[...], preferred_element_type=jnp.float32)
        + b1_ref[...], 0.0)
    h = jnp.maximum(
        jnp.dot(h, w2_ref[...], preferred_element_type=jnp.float32)
        + b2_ref[...], 0.0)
    out_ref[...] = (jnp.sum(fop_ref[...], axis=1) + jnp.sum(h, axis=1)
                    + bias_ref[0, 0])


def _tc_call(s, q, fop, W1, b1, W2, b2, bias):
    bn = 512
    grid = (BATCH // bn,)
    return pl.pallas_call(
        _tc_body,
        grid=grid,
        in_specs=[
            pl.BlockSpec((bn, EMB), lambda i: (i, 0)),
            pl.BlockSpec((bn, EMB), lambda i: (i, 0)),
            pl.BlockSpec((bn, FIELD), lambda i: (i, 0)),
            pl.BlockSpec((EMB, D1), lambda i: (0, 0)),
            pl.BlockSpec((1, D1), lambda i: (0, 0)),
            pl.BlockSpec((D1, D2), lambda i: (0, 0)),
            pl.BlockSpec((1, D2), lambda i: (0, 0)),
            pl.BlockSpec((1, 1), lambda i: (0, 0)),
        ],
        out_specs=pl.BlockSpec((bn,), lambda i: (i,)),
        out_shape=jax.ShapeDtypeStruct((BATCH,), jnp.float32),
    )(s, q, fop, W1, b1, W2, b2, bias)


def kernel(Xi, Xv, fo_tables, so_tables, W1, b1, W2, b2, bias):
    idx = (Xi[:, :, 0].astype(jnp.int32)
           + (jnp.arange(FIELD, dtype=jnp.int32) * VOCAB)[None, :])
    gidx = (idx >> 2).reshape(NW, NCHUNK, 2, 104)
    fidx = idx.reshape(NW, NCHUNK, 2, 104)
    off = ((idx & 3) << 5).reshape(NW, FPW)
    xv_w = Xv.reshape(NW, FPW)
    so_fold = so_tables.reshape(FIELD * VOCAB // 4, 4 * EMB)
    fo_flat = fo_tables.reshape(FIELD * VOCAB)
    s, q, fop = _sc_kernel(gidx, fidx, off, xv_w, so_fold, fo_flat)
    return _tc_call(s, q, fop.reshape(BATCH, FIELD), W1,
                    b1.reshape(1, D1), W2, b2.reshape(1, D2),
                    bias.reshape(1, 1))


# speedup vs baseline: 1.1927x; 1.1927x over previous
import functools

import jax
import jax.numpy as jnp
from jax import lax
from jax.experimental import pallas as pl
from jax.experimental.pallas import tpu as pltpu
from jax.experimental.pallas import tpu_sc as plsc

FIELD = 26
VOCAB = 100000
EMB = 32
BATCH = 4096
D1 = 32
D2 = 32

NW = 32
RPW = BATCH // NW
FPW = RPW * FIELD
CROWS = 4
GCH = CROWS * FIELD
NCHUNK = RPW // CROWS
NBUF = 3


def _sc_main_body(gidx_hbm, off_hbm, xv_hbm, so_hbm,
                  s_hbm, q_hbm,
                  gidx_v, off_v, xv_v, s_v, q_v,
                  emb0, emb1, emb2, sem0, sem1, sem2):
    wid = lax.axis_index("s") * 2 + lax.axis_index("c")
    bufs = (emb0, emb1, emb2)
    sems = (sem0, sem1, sem2)

    pltpu.sync_copy(gidx_hbm.at[wid], gidx_v)
    pltpu.sync_copy(off_hbm.at[wid], off_v)
    pltpu.sync_copy(xv_hbm.at[wid], xv_v)

    def fire(c):
        return pltpu.async_copy(so_hbm.at[gidx_v.at[c, 0]],
                                bufs[c % NBUF].at[0], sems[c % NBUF])

    descs = {c: fire(c) for c in range(NBUF)}

    def make_row_body(c):
        b = bufs[c % NBUF]

        def row_body(r, carry):
            base = r * FIELD
            gbase = c * GCH + base
            xa = xv_v[0, pl.ds(gbase, 16)]
            xb = xv_v[0, pl.ds(gbase + FIELD - 16, 16)]
            oa = off_v[0, pl.ds(gbase, 16)]
            ob = off_v[0, pl.ds(gbase + FIELD - 16, 16)]
            s0 = jnp.zeros((16,), jnp.float32)
            s1 = jnp.zeros((16,), jnp.float32)
            q0 = jnp.zeros((16,), jnp.float32)
            q1 = jnp.zeros((16,), jnp.float32)
            for f in range(FIELD):
                if f < 16:
                    xv, off = xa[f], oa[f]
                else:
                    xv, off = xb[f - (FIELD - 16)], ob[f - (FIELD - 16)]
                t0 = b[0, base + f, pl.ds(off, 16)] * xv
                t1 = b[0, base + f, pl.ds(off + 16, 16)] * xv
                s0 += t0
                s1 += t1
                q0 += t0 * t0
                q1 += t1 * t1
            row = (c * CROWS + r) * EMB
            s_v[pl.ds(row, 16)] = s0
            s_v[pl.ds(row + 16, 16)] = s1
            q_v[pl.ds(row, 16)] = q0
            q_v[pl.ds(row + 16, 16)] = q1
            return carry

        return row_body

    for c in range(NCHUNK):
        descs.pop(c).wait()
        lax.fori_loop(0, CROWS, make_row_body(c), 0)
        if c + NBUF < NCHUNK:
            descs[c + NBUF] = fire(c + NBUF)

    pltpu.sync_copy(s_v, s_hbm.at[wid])
    pltpu.sync_copy(q_v, q_hbm.at[wid])


_sc_main = functools.partial(
    pl.kernel,
    out_type=[
        jax.ShapeDtypeStruct((NW, RPW * EMB), jnp.float32),
        jax.ShapeDtypeStruct((NW, RPW * EMB), jnp.float32),
    ],
    mesh=plsc.VectorSubcoreMesh(core_axis_name="c", subcore_axis_name="s"),
    compiler_params=pltpu.CompilerParams(use_tc_tiling_on_sc=True),
    scratch_types=[
        pltpu.VMEM((NCHUNK, 1, 128), jnp.int32),
        pltpu.VMEM((1, FPW), jnp.int32),
        pltpu.VMEM((1, FPW), jnp.float32),
        pltpu.VMEM((RPW * EMB,), jnp.float32),
        pltpu.VMEM((RPW * EMB,), jnp.float32),
        pltpu.VMEM((1, 128, 128), jnp.float32),
        pltpu.VMEM((1, 128, 128), jnp.float32),
        pltpu.VMEM((1, 128, 128), jnp.float32),
        pltpu.SemaphoreType.DMA,
        pltpu.SemaphoreType.DMA,
        pltpu.SemaphoreType.DMA,
    ],
)(_sc_main_body)


def _sc_fo_body(fidx_hbm, xv_hbm, fo_hbm,
                fop_hbm,
                fidx_v, xv_v, fo_v, sem_f):
    wid = lax.axis_index("s") * 2 + lax.axis_index("c")

    pltpu.sync_copy(fidx_hbm.at[wid], fidx_v)
    pltpu.sync_copy(xv_hbm.at[wid], xv_v)
    cps = []
    for j in range(FPW // 128):
        cps.append(pltpu.async_copy(
            fo_hbm.at[fidx_v.at[j]], fo_v.at[pl.ds(j * 128, 128)], sem_f))
    for d in cps:
        d.wait()

    def fo_body(k, carry):
        off = k * 16
        fo_v[pl.ds(off, 16)] = fo_v[pl.ds(off, 16)] * xv_v[pl.ds(off, 16)]
        return carry

    lax.fori_loop(0, FPW // 16, fo_body, 0)
    pltpu.sync_copy(fo_v, fop_hbm.at[wid])


_sc_fo = functools.partial(
    pl.kernel,
    out_type=[jax.ShapeDtypeStruct((NW, FPW), jnp.float32)],
    mesh=plsc.VectorSubcoreMesh(core_axis_name="c", subcore_axis_name="s"),
    compiler_params=pltpu.CompilerParams(use_tc_tiling_on_sc=False),
    scratch_types=[
        pltpu.VMEM((FPW // 128, 128), jnp.int32),
        pltpu.VMEM((FPW,), jnp.float32),
        pltpu.VMEM((FPW,), jnp.float32),
        pltpu.SemaphoreType.DMA,
    ],
)(_sc_fo_body)


QV = VOCAB // 4


def _fold_body(so_ref, eye_ref, out_ref):
    x = so_ref[0]
    for k in range(4):
        yk = lax.dot_general(x[:, QV * k:QV * (k + 1)], eye_ref[...],
                             (((0,), (0,)), ((), ())),
                             preferred_element_type=jnp.float32)
        out_ref[:, EMB * k:EMB * (k + 1)] = yk


def _fold_call(so_t, eye):
    return pl.pallas_call(
        _fold_body,
        grid=(FIELD,),
        in_specs=[
            pl.BlockSpec((1, EMB, VOCAB), lambda f: (f, 0, 0),
                         pipeline_mode=pl.Buffered(buffer_count=2)),
            pl.BlockSpec((EMB, EMB), lambda f: (0, 0)),
        ],
        out_specs=pl.BlockSpec((QV, 4 * EMB), lambda f: (f, 0),
                               pipeline_mode=pl.Buffered(buffer_count=1)),
        out_shape=jax.ShapeDtypeStruct((FIELD * QV, 4 * EMB), jnp.float32),
    )(so_t, eye)


def _tc_body(s_ref, q_ref, fop_ref, w1_ref, b1_ref, w2_ref, b2_ref,
             bias_ref, out_ref):
    s = s_ref[...]
    x = 0.5 * (s * s - q_ref[...])
    h = jnp.maximum(
        jnp.dot(x, w1_ref[...], preferred_element_type=jnp.float32)
        + b1_ref[...], 0.0)
    h = jnp.maximum(
        jnp.dot(h, w2_ref[...], preferred_element_type=jnp.float32)
        + b2_ref[...], 0.0)
    out_ref[...] = (jnp.sum(fop_ref[...], axis=1) + jnp.sum(h, axis=1)
                    + bias_ref[0, 0])


def _tc_call(s, q, fop, W1, b1, W2, b2, bias):
    bn = 512
    grid = (BATCH // bn,)
    return pl.pallas_call(
        _tc_body,
        grid=grid,
        in_specs=[
            pl.BlockSpec((bn, EMB), lambda i: (i, 0)),
            pl.BlockSpec((bn, EMB), lambda i: (i, 0)),
            pl.BlockSpec((bn, FIELD), lambda i: (i, 0)),
            pl.BlockSpec((EMB, D1), lambda i: (0, 0)),
            pl.BlockSpec((1, D1), lambda i: (0, 0)),
            pl.BlockSpec((D1, D2), lambda i: (0, 0)),
            pl.BlockSpec((1, D2), lambda i: (0, 0)),
            pl.BlockSpec((1, 1), lambda i: (0, 0)),
        ],
        out_specs=pl.BlockSpec((bn,), lambda i: (i,)),
        out_shape=jax.ShapeDtypeStruct((BATCH,), jnp.float32),
    )(s, q, fop, W1, b1, W2, b2, bias)


def kernel(Xi, Xv, fo_tables, so_tables, W1, b1, W2, b2, bias):
    v = Xi[:, :, 0].astype(jnp.int32)
    idx = v + (jnp.arange(FIELD, dtype=jnp.int32) * VOCAB)[None, :]
    gidx = ((jnp.arange(FIELD, dtype=jnp.int32) * QV)[None, :]
            + v % QV).reshape(NW, NCHUNK, 1, GCH)
    gidx = jnp.pad(gidx, ((0, 0), (0, 0), (0, 0), (0, 128 - GCH)),
                   mode="edge")
    off = ((v // QV) << 5).reshape(NW, 1, FPW)
    xv_w = Xv.reshape(NW, 1, FPW)
    so_t = jnp.transpose(so_tables, (0, 2, 1))
    so_fold = _fold_call(so_t, jnp.eye(EMB, dtype=jnp.float32))
    s, q = _sc_main(gidx, off, xv_w, so_fold)
    fidx = idx.reshape(NW, FPW // 128, 128)
    fop, = _sc_fo(fidx, Xv.reshape(NW, FPW), fo_tables.reshape(FIELD * VOCAB))
    return _tc_call(s.reshape(BATCH, EMB), q.reshape(BATCH, EMB),
                    fop.reshape(BATCH, FIELD), W1,
                    b1.reshape(1, D1), W2, b2.reshape(1, D2),
                    bias.reshape(1, 1))


# speedup vs baseline: 1.7801x; 1.4926x over previous
import functools

import jax
import jax.numpy as jnp
from jax import lax
from jax.experimental import pallas as pl
from jax.experimental.pallas import tpu as pltpu
from jax.experimental.pallas import tpu_sc as plsc

FIELD = 26
VOCAB = 100000
EMB = 32
BATCH = 4096
D1 = 32
D2 = 32

NW = 32
RPW = BATCH // NW
FPW = RPW * FIELD
CROWS = 4
GCH = CROWS * FIELD
NCHUNK = RPW // CROWS
NBUF = 3


def _sc_main_body(gidx_hbm, off_hbm, xv_hbm, so_hbm,
                  s_hbm, q_hbm,
                  gidx_v, off_v, xv_v, s_v, q_v,
                  emb0, emb1, emb2, sem0, sem1, sem2):
    wid = lax.axis_index("s") * 2 + lax.axis_index("c")
    bufs = (emb0, emb1, emb2)
    sems = (sem0, sem1, sem2)

    pltpu.sync_copy(gidx_hbm.at[wid], gidx_v)
    pltpu.sync_copy(off_hbm.at[wid], off_v)
    pltpu.sync_copy(xv_hbm.at[wid], xv_v)

    def fire(c):
        return pltpu.async_copy(so_hbm.at[gidx_v.at[c, 0]],
                                bufs[c % NBUF].at[0], sems[c % NBUF])

    descs = {c: fire(c) for c in range(NBUF)}

    def make_row_body(c):
        b = bufs[c % NBUF]

        def row_body(r, carry):
            base = r * FIELD
            gbase = c * GCH + base
            xa = xv_v[0, pl.ds(gbase, 16)]
            xb = xv_v[0, pl.ds(gbase + FIELD - 16, 16)]
            oa = off_v[0, pl.ds(gbase, 16)]
            ob = off_v[0, pl.ds(gbase + FIELD - 16, 16)]
            s0 = jnp.zeros((16,), jnp.float32)
            s1 = jnp.zeros((16,), jnp.float32)
            q0 = jnp.zeros((16,), jnp.float32)
            q1 = jnp.zeros((16,), jnp.float32)
            for f in range(FIELD):
                if f < 16:
                    xv, off = xa[f], oa[f]
                else:
                    xv, off = xb[f - (FIELD - 16)], ob[f - (FIELD - 16)]
                t0 = b[0, base + f, pl.ds(off, 16)] * xv
                t1 = b[0, base + f, pl.ds(off + 16, 16)] * xv
                s0 += t0
                s1 += t1
                q0 += t0 * t0
                q1 += t1 * t1
            row = (c * CROWS + r) * EMB
            s_v[pl.ds(row, 16)] = s0
            s_v[pl.ds(row + 16, 16)] = s1
            q_v[pl.ds(row, 16)] = q0
            q_v[pl.ds(row + 16, 16)] = q1
            return carry

        return row_body

    for c in range(NCHUNK):
        descs.pop(c).wait()
        lax.fori_loop(0, CROWS, make_row_body(c), 0)
        if c + NBUF < NCHUNK:
            descs[c + NBUF] = fire(c + NBUF)

    pltpu.sync_copy(s_v, s_hbm.at[wid])
    pltpu.sync_copy(q_v, q_hbm.at[wid])


_sc_main = functools.partial(
    pl.kernel,
    out_type=[
        jax.ShapeDtypeStruct((NW, RPW * EMB), jnp.float32),
        jax.ShapeDtypeStruct((NW, RPW * EMB), jnp.float32),
    ],
    mesh=plsc.VectorSubcoreMesh(core_axis_name="c", subcore_axis_name="s"),
    compiler_params=pltpu.CompilerParams(use_tc_tiling_on_sc=True),
    scratch_types=[
        pltpu.VMEM((NCHUNK, 1, 128), jnp.int32),
        pltpu.VMEM((1, FPW), jnp.int32),
        pltpu.VMEM((1, FPW), jnp.float32),
        pltpu.VMEM((RPW * EMB,), jnp.float32),
        pltpu.VMEM((RPW * EMB,), jnp.float32),
        pltpu.VMEM((1, 128, 128), jnp.float32),
        pltpu.VMEM((1, 128, 128), jnp.float32),
        pltpu.VMEM((1, 128, 128), jnp.float32),
        pltpu.SemaphoreType.DMA,
        pltpu.SemaphoreType.DMA,
        pltpu.SemaphoreType.DMA,
    ],
)(_sc_main_body)


def _sc_fo_body(fidx_hbm, xv_hbm, fo_hbm,
                fop_hbm,
                fidx_v, xv_v, fo_v, sem_f):
    wid = lax.axis_index("s") * 2 + lax.axis_index("c")

    pltpu.sync_copy(fidx_hbm.at[wid], fidx_v)
    pltpu.sync_copy(xv_hbm.at[wid], xv_v)
    cps = []
    for j in range(FPW // 128):
        cps.append(pltpu.async_copy(
            fo_hbm.at[fidx_v.at[j]], fo_v.at[pl.ds(j * 128, 128)], sem_f))
    for d in cps:
        d.wait()

    def fo_body(k, carry):
        off = k * 16
        fo_v[pl.ds(off, 16)] = fo_v[pl.ds(off, 16)] * xv_v[pl.ds(off, 16)]
        return carry

    lax.fori_loop(0, FPW // 16, fo_body, 0)
    pltpu.sync_copy(fo_v, fop_hbm.at[wid])


_sc_fo = functools.partial(
    pl.kernel,
    out_type=[jax.ShapeDtypeStruct((NW, FPW), jnp.float32)],
    mesh=plsc.VectorSubcoreMesh(core_axis_name="c", subcore_axis_name="s"),
    compiler_params=pltpu.CompilerParams(use_tc_tiling_on_sc=False),
    scratch_types=[
        pltpu.VMEM((FPW // 128, 128), jnp.int32),
        pltpu.VMEM((FPW,), jnp.float32),
        pltpu.VMEM((FPW,), jnp.float32),
        pltpu.SemaphoreType.DMA,
    ],
)(_sc_fo_body)


QV = 25088


def _fold_body(so_ref, sel_ref, out_ref):
    x = so_ref[0]
    acc = None
    for k in range(4):
        yk = lax.dot_general(x[:, QV * k:QV * (k + 1)], sel_ref[k],
                             (((0,), (0,)), ((), ())),
                             preferred_element_type=jnp.float32)
        acc = yk if acc is None else acc + yk
    out_ref[...] = acc


def _fold_call(so_t, sel):
    return pl.pallas_call(
        _fold_body,
        grid=(FIELD,),
        in_specs=[
            pl.BlockSpec((1, EMB, 4 * QV), lambda f: (f, 0, 0),
                         pipeline_mode=pl.Buffered(buffer_count=2)),
            pl.BlockSpec((4, EMB, 4 * EMB), lambda f: (0, 0, 0)),
        ],
        out_specs=pl.BlockSpec((QV, 4 * EMB), lambda f: (f, 0),
                               pipeline_mode=pl.Buffered(buffer_count=1)),
        out_shape=jax.ShapeDtypeStruct((FIELD * QV, 4 * EMB), jnp.float32),
    )(so_t, sel)


def _tc_body(s_ref, q_ref, fop_ref, w1_ref, b1_ref, w2_ref, b2_ref,
             bias_ref, out_ref):
    s = s_ref[...]
    x = 0.5 * (s * s - q_ref[...])
    h = jnp.maximum(
        jnp.dot(x, w1_ref[...], preferred_element_type=jnp.float32)
        + b1_ref[...], 0.0)
    h = jnp.maximum(
        jnp.dot(h, w2_ref[...], preferred_element_type=jnp.float32)
        + b2_ref[...], 0.0)
    out_ref[...] = (jnp.sum(fop_ref[...], axis=1) + jnp.sum(h, axis=1)
                    + bias_ref[0, 0])


def _tc_call(s, q, fop, W1, b1, W2, b2, bias):
    bn = 512
    grid = (BATCH // bn,)
    return pl.pallas_call(
        _tc_body,
        grid=grid,
        in_specs=[
            pl.BlockSpec((bn, EMB), lambda i: (i, 0)),
            pl.BlockSpec((bn, EMB), lambda i: (i, 0)),
            pl.BlockSpec((bn, FIELD), lambda i: (i, 0)),
            pl.BlockSpec((EMB, D1), lambda i: (0, 0)),
            pl.BlockSpec((1, D1), lambda i: (0, 0)),
            pl.BlockSpec((D1, D2), lambda i: (0, 0)),
            pl.BlockSpec((1, D2), lambda i: (0, 0)),
            pl.BlockSpec((1, 1), lambda i: (0, 0)),
        ],
        out_specs=pl.BlockSpec((bn,), lambda i: (i,)),
        out_shape=jax.ShapeDtypeStruct((BATCH,), jnp.float32),
    )(s, q, fop, W1, b1, W2, b2, bias)


def kernel(Xi, Xv, fo_tables, so_tables, W1, b1, W2, b2, bias):
    v = Xi[:, :, 0].astype(jnp.int32)
    idx = v + (jnp.arange(FIELD, dtype=jnp.int32) * VOCAB)[None, :]
    gidx = ((jnp.arange(FIELD, dtype=jnp.int32) * QV)[None, :]
            + v % QV).reshape(NW, NCHUNK, 1, GCH)
    gidx = jnp.pad(gidx, ((0, 0), (0, 0), (0, 0), (0, 128 - GCH)),
                   mode="edge")
    off = ((v // QV) << 5).reshape(NW, 1, FPW)
    xv_w = Xv.reshape(NW, 1, FPW)
    so_t = jnp.transpose(so_tables, (0, 2, 1))
    sel = jnp.stack([
        jnp.pad(jnp.eye(EMB, dtype=jnp.float32),
                ((0, 0), (EMB * k, 4 * EMB - EMB * (k + 1))))
        for k in range(4)])
    so_fold = _fold_call(so_t, sel)
    s, q = _sc_main(gidx, off, xv_w, so_fold)
    fidx = idx.reshape(NW, FPW // 128, 128)
    fop, = _sc_fo(fidx, Xv.reshape(NW, FPW), fo_tables.reshape(FIELD * VOCAB))
    return _tc_call(s.reshape(BATCH, EMB), q.reshape(BATCH, EMB),
                    fop.reshape(BATCH, FIELD), W1,
                    b1.reshape(1, D1), W2, b2.reshape(1, D2),
                    bias.reshape(1, 1))


# speedup vs baseline: 1.9444x; 1.0923x over previous
import functools

import jax
import jax.numpy as jnp
from jax import lax
from jax.experimental import pallas as pl
from jax.experimental.pallas import tpu as pltpu
from jax.experimental.pallas import tpu_sc as plsc

FIELD = 26
VOCAB = 100000
EMB = 32
BATCH = 4096
D1 = 32
D2 = 32

NW = 32
RPW = BATCH // NW
FPW = RPW * FIELD
CROWS = 4
GCH = CROWS * FIELD
NCHUNK = RPW // CROWS
NBUF = 3


def _sc_main_body(gidx_hbm, off_hbm, xv_hbm, so_hbm,
                  s_hbm, q_hbm,
                  gidx_v, off_v, xv_v, s_v, q_v,
                  emb0, emb1, emb2, sem0, sem1, sem2):
    wid = lax.axis_index("s") * 2 + lax.axis_index("c")
    bufs = (emb0, emb1, emb2)
    sems = (sem0, sem1, sem2)

    pltpu.sync_copy(gidx_hbm.at[wid], gidx_v)
    pltpu.sync_copy(off_hbm.at[wid], off_v)
    pltpu.sync_copy(xv_hbm.at[wid], xv_v)

    def fire(c):
        return pltpu.async_copy(so_hbm.at[gidx_v.at[c, 0]],
                                bufs[c % NBUF].at[0], sems[c % NBUF])

    descs = {c: fire(c) for c in range(NBUF)}

    def make_row_body(c):
        b = bufs[c % NBUF]

        def row_body(r, carry):
            base = r * FIELD
            gbase = c * GCH + base
            xa = xv_v[0, pl.ds(gbase, 16)]
            xb = xv_v[0, pl.ds(gbase + FIELD - 16, 16)]
            oa = off_v[0, pl.ds(gbase, 16)]
            ob = off_v[0, pl.ds(gbase + FIELD - 16, 16)]
            s0 = jnp.zeros((16,), jnp.float32)
            s1 = jnp.zeros((16,), jnp.float32)
            q0 = jnp.zeros((16,), jnp.float32)
            q1 = jnp.zeros((16,), jnp.float32)
            for f in range(FIELD):
                if f < 16:
                    xv, off = xa[f], oa[f]
                else:
                    xv, off = xb[f - (FIELD - 16)], ob[f - (FIELD - 16)]
                t0 = b[0, base + f, pl.ds(off, 16)] * xv
                t1 = b[0, base + f, pl.ds(off + 16, 16)] * xv
                s0 += t0
                s1 += t1
                q0 += t0 * t0
                q1 += t1 * t1
            row = (c * CROWS + r) * EMB
            s_v[pl.ds(row, 16)] = s0
            s_v[pl.ds(row + 16, 16)] = s1
            q_v[pl.ds(row, 16)] = q0
            q_v[pl.ds(row + 16, 16)] = q1
            return carry

        return row_body

    for c in range(NCHUNK):
        descs.pop(c).wait()
        lax.fori_loop(0, CROWS, make_row_body(c), 0)
        if c + NBUF < NCHUNK:
            descs[c + NBUF] = fire(c + NBUF)

    pltpu.sync_copy(s_v, s_hbm.at[wid])
    pltpu.sync_copy(q_v, q_hbm.at[wid])


_sc_main = functools.partial(
    pl.kernel,
    out_type=[
        jax.ShapeDtypeStruct((NW, RPW * EMB), jnp.float32),
        jax.ShapeDtypeStruct((NW, RPW * EMB), jnp.float32),
    ],
    mesh=plsc.VectorSubcoreMesh(core_axis_name="c", subcore_axis_name="s"),
    compiler_params=pltpu.CompilerParams(use_tc_tiling_on_sc=True),
    scratch_types=[
        pltpu.VMEM((NCHUNK, 1, 128), jnp.int32),
        pltpu.VMEM((1, FPW), jnp.int32),
        pltpu.VMEM((1, FPW), jnp.float32),
        pltpu.VMEM((RPW * EMB,), jnp.float32),
        pltpu.VMEM((RPW * EMB,), jnp.float32),
        pltpu.VMEM((1, 128, 128), jnp.float32),
        pltpu.VMEM((1, 128, 128), jnp.float32),
        pltpu.VMEM((1, 128, 128), jnp.float32),
        pltpu.SemaphoreType.DMA,
        pltpu.SemaphoreType.DMA,
        pltpu.SemaphoreType.DMA,
    ],
)(_sc_main_body)


def _sc_fo_body(fidx_hbm, xv_hbm, fo_hbm,
                fop_hbm,
                fidx_v, xv_v, fo_v, sem_f):
    wid = lax.axis_index("s") * 2 + lax.axis_index("c")

    pltpu.sync_copy(fidx_hbm.at[wid], fidx_v)
    pltpu.sync_copy(xv_hbm.at[wid], xv_v)
    cps = []
    for j in range(FPW // 128):
        cps.append(pltpu.async_copy(
            fo_hbm.at[fidx_v.at[j]], fo_v.at[pl.ds(j * 128, 128)], sem_f))
    for d in cps:
        d.wait()

    def fo_body(k, carry):
        off = k * 16
        fo_v[pl.ds(off, 16)] = fo_v[pl.ds(off, 16)] * xv_v[pl.ds(off, 16)]
        return carry

    lax.fori_loop(0, FPW // 16, fo_body, 0)
    pltpu.sync_copy(fo_v, fop_hbm.at[wid])


_sc_fo = functools.partial(
    pl.kernel,
    out_type=[jax.ShapeDtypeStruct((NW, FPW), jnp.float32)],
    mesh=plsc.VectorSubcoreMesh(core_axis_name="c", subcore_axis_name="s"),
    compiler_params=pltpu.CompilerParams(use_tc_tiling_on_sc=False),
    scratch_types=[
        pltpu.VMEM((FPW // 128, 128), jnp.int32),
        pltpu.VMEM((FPW,), jnp.float32),
        pltpu.VMEM((FPW,), jnp.float32),
        pltpu.SemaphoreType.DMA,
    ],
)(_sc_fo_body)


QV = 25088


def _fold_body(so_ref, sel_ref, out_ref):
    r = pl.program_id(1)
    x = so_ref[0]
    acc = None
    for kk in range(2):
        yk = lax.dot_general(x[:, QV * kk:QV * (kk + 1)],
                             sel_ref[2 * r + kk],
                             (((0,), (0,)), ((), ())),
                             preferred_element_type=jnp.float32)
        acc = yk if acc is None else acc + yk

    @pl.when(r == 0)
    def _():
        out_ref[...] = acc

    @pl.when(r != 0)
    def _():
        out_ref[...] = out_ref[...] + acc


def _fold_call(so_t, sel):
    return pl.pallas_call(
        _fold_body,
        grid=(FIELD, 2),
        in_specs=[
            pl.BlockSpec((1, EMB, 2 * QV), lambda f, r: (f, 0, r),
                         pipeline_mode=pl.Buffered(buffer_count=2)),
            pl.BlockSpec((4, EMB, 4 * EMB), lambda f, r: (0, 0, 0)),
        ],
        out_specs=pl.BlockSpec((QV, 4 * EMB), lambda f, r: (f, 0),
                               pipeline_mode=pl.Buffered(buffer_count=2)),
        out_shape=jax.ShapeDtypeStruct((FIELD * QV, 4 * EMB), jnp.float32),
    )(so_t, sel)


def _tc_body(s_ref, q_ref, fop_ref, w1_ref, b1_ref, w2_ref, b2_ref,
             bias_ref, out_ref):
    s = s_ref[...]
    x = 0.5 * (s * s - q_ref[...])
    h = jnp.maximum(
        jnp.dot(x, w1_ref[...], preferred_element_type=jnp.float32)
        + b1_ref[...], 0.0)
    h = jnp.maximum(
        jnp.dot(h, w2_ref[...], preferred_element_type=jnp.float32)
        + b2_ref[...], 0.0)
    out_ref[...] = (jnp.sum(fop_ref[...], axis=1) + jnp.sum(h, axis=1)
                    + bias_ref[0, 0])


def _tc_call(s, q, fop, W1, b1, W2, b2, bias):
    bn = 512
    grid = (BATCH // bn,)
    return pl.pallas_call(
        _tc_body,
        grid=grid,
        in_specs=[
            pl.BlockSpec((bn, EMB), lambda i: (i, 0)),
            pl.BlockSpec((bn, EMB), lambda i: (i, 0)),
            pl.BlockSpec((bn, FIELD), lambda i: (i, 0)),
            pl.BlockSpec((EMB, D1), lambda i: (0, 0)),
            pl.BlockSpec((1, D1), lambda i: (0, 0)),
            pl.BlockSpec((D1, D2), lambda i: (0, 0)),
            pl.BlockSpec((1, D2), lambda i: (0, 0)),
            pl.BlockSpec((1, 1), lambda i: (0, 0)),
        ],
        out_specs=pl.BlockSpec((bn,), lambda i: (i,)),
        out_shape=jax.ShapeDtypeStruct((BATCH,), jnp.float32),
    )(s, q, fop, W1, b1, W2, b2, bias)


def kernel(Xi, Xv, fo_tables, so_tables, W1, b1, W2, b2, bias):
    v = Xi[:, :, 0].astype(jnp.int32)
    idx = v + (jnp.arange(FIELD, dtype=jnp.int32) * VOCAB)[None, :]
    gidx = ((jnp.arange(FIELD, dtype=jnp.int32) * QV)[None, :]
            + v % QV).reshape(NW, NCHUNK, 1, GCH)
    gidx = jnp.pad(gidx, ((0, 0), (0, 0), (0, 0), (0, 128 - GCH)),
                   mode="edge")
    off = ((v // QV) << 5).reshape(NW, 1, FPW)
    xv_w = Xv.reshape(NW, 1, FPW)
    so_t = jnp.transpose(so_tables, (0, 2, 1))
    sel = jnp.stack([
        jnp.pad(jnp.eye(EMB, dtype=jnp.float32),
                ((0, 0), (EMB * k, 4 * EMB - EMB * (k + 1))))
        for k in range(4)])
    so_fold = _fold_call(so_t, sel)
    s, q = _sc_main(gidx, off, xv_w, so_fold)
    fidx = idx.reshape(NW, FPW // 128, 128)
    fop, = _sc_fo(fidx, Xv.reshape(NW, FPW), fo_tables.reshape(FIELD * VOCAB))
    return _tc_call(s.reshape(BATCH, EMB), q.reshape(BATCH, EMB),
                    fop.reshape(BATCH, FIELD), W1,
                    b1.reshape(1, D1), W2, b2.reshape(1, D2),
                    bias.reshape(1, 1))
